# baseline (speedup 1.0000x reference)
import functools

import jax
import jax.numpy as jnp
from jax import lax
from jax.experimental import pallas as pl
from jax.experimental.pallas import tpu as pltpu
from jax.experimental.pallas import tpu_sc as plsc

_B = 64
_S = 64
_H = 2048
_NC = 2
_NS = 16
_NW = _NC * _NS
_RPW = (_B * _S) // _NW
_CHUNK = 16
_NT = _RPW // _CHUNK


def _norm_body(prev_ref, w1_ref, new_ref, norms_ref, old0_ref,
               l2_ref, k_ref, idx_ref, mix_ref):
    pred = jnp.tanh(
        lax.dot_general(
            prev_ref[...], w1_ref[...], (((1,), (0,)), ((), ())),
            preferred_element_type=jnp.float32,
            precision=lax.Precision.DEFAULT,
        )
    )
    diff = pred - new_ref[...]
    l2 = jnp.sqrt(jnp.sum(diff * diff, axis=1, keepdims=True))
    n0 = norms_ref[:, 0:1]
    rep = l2 > n0
    v = jnp.where(rep, l2, n0)
    k = jnp.sum((norms_ref[:, 1:] < v).astype(jnp.int32), axis=1, keepdims=True)
    l2_ref[...] = l2
    k_ref[...] = k
    j = lax.broadcasted_iota(jnp.int32, (_B, _S), 1)
    brow = lax.broadcasted_iota(jnp.int32, (_B, _S), 0) * _S
    src = brow + j + (j < k).astype(jnp.int32)
    idx_ref[...] = jnp.where(j == k, brow, src)
    mix_ref[...] = jnp.where(rep, new_ref[...], old0_ref[...])


def _sc_gather_body(states_ref, idx_ref, out_ref,
                    idx0, idx1, buf0, buf1, sem0, sem1):
    wid = lax.axis_index("s") * _NC + lax.axis_index("c")
    idxs = (idx0, idx1)
    bufs = (buf0, buf1)
    sems = (sem0, sem1)
    handles = [None, None]

    def start(t):
        base = wid * _RPW + t * _CHUNK
        pltpu.sync_copy(idx_ref.at[pl.ds(base, _CHUNK)], idxs[t % 2])
        handles[t % 2] = pltpu.async_copy(
            states_ref.at[idxs[t % 2]], bufs[t % 2], sems[t % 2])

    def finish(t):
        base = wid * _RPW + t * _CHUNK
        handles[t % 2].wait()
        pltpu.sync_copy(bufs[t % 2], out_ref.at[pl.ds(base, _CHUNK)])

    start(0)
    for t in range(1, _NT):
        start(t)
        finish(t - 1)
    finish(_NT - 1)


@functools.cache
def _sc_gather():
    return pl.kernel(
        _sc_gather_body,
        out_type=jax.ShapeDtypeStruct((_B * _S, _H), jnp.float32),
        mesh=plsc.VectorSubcoreMesh(
            core_axis_name="c", subcore_axis_name="s",
            num_cores=_NC, num_subcores=_NS),
        scratch_types=[
            pltpu.VMEM((_CHUNK,), jnp.int32),
            pltpu.VMEM((_CHUNK,), jnp.int32),
            pltpu.VMEM((_CHUNK, _H), jnp.float32),
            pltpu.VMEM((_CHUNK, _H), jnp.float32),
            pltpu.SemaphoreType.DMA,
            pltpu.SemaphoreType.DMA,
        ],
    )


def _patch_body(karr_ref, mix_ref, outin_ref, out_ref):
    del karr_ref, outin_ref
    out_ref[0] = mix_ref[...]


def kernel(old_states, new_state, prev_state, old_norms, w1):
    l2, k, idx, mix = pl.pallas_call(
        _norm_body,
        out_shape=[
            jax.ShapeDtypeStruct((_B, 1), jnp.float32),
            jax.ShapeDtypeStruct((_B, 1), jnp.int32),
            jax.ShapeDtypeStruct((_B, _S), jnp.int32),
            jax.ShapeDtypeStruct((_B, _H), jnp.float32),
        ],
    )(prev_state, w1, new_state, old_norms, old_states[:, 0, :])

    gathered = _sc_gather()(old_states.reshape(_B * _S, _H), idx.reshape(_B * _S))

    out = pl.pallas_call(
        _patch_body,
        grid_spec=pltpu.PrefetchScalarGridSpec(
            num_scalar_prefetch=1,
            grid=(_B,),
            in_specs=[
                pl.BlockSpec((1, 1, _H), lambda b, karr: (b, 0, 0)),
                pl.BlockSpec(memory_space=pl.ANY),
            ],
            out_specs=pl.BlockSpec(
                (1, 1, 1, _H), lambda b, karr: (b, karr[b], 0, 0)),
        ),
        out_shape=jax.ShapeDtypeStruct((_B, _S, 1, _H), jnp.float32),
        input_output_aliases={2: 0},
    )(k.reshape(_B), mix.reshape(_B, 1, _H), gathered.reshape(_B, _S, 1, _H))

    return out.reshape(_B, _S, _H), l2


# speedup vs baseline: 3.9336x; 3.9336x over previous
import functools

import jax
import jax.numpy as jnp
from jax import lax
from jax.experimental import pallas as pl
from jax.experimental.pallas import tpu as pltpu
from jax.experimental.pallas import tpu_sc as plsc

_B = 64
_S = 64
_H = 2048
_NC = 2
_NS = 16
_NW = _NC * _NS
_BPW = _B // _NW
_RPW = (_B * _S) // _NW
_CHUNK = 16
_NT = _RPW // _CHUNK


def _norm_body(prev_ref, w1_ref, new_ref, norms_ref, old0_ref,
               l2_ref, idx_ref, mix_ref, dst_ref):
    pred = jnp.tanh(
        lax.dot_general(
            prev_ref[...], w1_ref[...], (((1,), (0,)), ((), ())),
            preferred_element_type=jnp.float32,
            precision=lax.Precision.DEFAULT,
        )
    )
    diff = pred - new_ref[...]
    l2 = jnp.sqrt(jnp.sum(diff * diff, axis=1, keepdims=True))
    n0 = norms_ref[:, 0:1]
    rep = l2 > n0
    v = jnp.where(rep, l2, n0)
    k = jnp.sum((norms_ref[:, 1:] < v).astype(jnp.int32), axis=1, keepdims=True)
    l2_ref[...] = l2
    j = lax.broadcasted_iota(jnp.int32, (_B, _S), 1)
    brow = lax.broadcasted_iota(jnp.int32, (_B, _S), 0) * _S
    src = brow + j + (j < k).astype(jnp.int32)
    idx_ref[...] = jnp.where(j == k, brow, src)
    mix_ref[...] = jnp.where(rep, new_ref[...], old0_ref[...])
    dst_ref[...] = (brow[:, 0:1] + k).reshape(_NW, _BPW)


def _sc_reorder_body(states_ref, idx_ref, mix_ref, dst_ref, out_ref,
                     idx0, idx1, buf0, buf1, mixbuf, didx, sem0, sem1):
    wid = lax.axis_index("s") * _NC + lax.axis_index("c")
    idxs = (idx0, idx1)
    bufs = (buf0, buf1)
    sems = (sem0, sem1)
    handles = [None, None]

    def start(t):
        base = wid * _RPW + t * _CHUNK
        pltpu.sync_copy(idx_ref.at[pl.ds(base, _CHUNK)], idxs[t % 2])
        handles[t % 2] = pltpu.async_copy(
            states_ref.at[idxs[t % 2]], bufs[t % 2], sems[t % 2])

    def finish(t):
        base = wid * _RPW + t * _CHUNK
        handles[t % 2].wait()
        pltpu.sync_copy(bufs[t % 2], out_ref.at[pl.ds(base, _CHUNK)])

    start(0)
    for t in range(1, _NT):
        start(t)
        finish(t - 1)
    finish(_NT - 1)

    pltpu.sync_copy(mix_ref.at[pl.ds(wid * _BPW, _BPW)], mixbuf)
    pltpu.sync_copy(dst_ref.at[wid], didx)
    pltpu.sync_copy(mixbuf, out_ref.at[didx])


@functools.cache
def _sc_reorder():
    return pl.kernel(
        _sc_reorder_body,
        out_type=jax.ShapeDtypeStruct((_B * _S, _H), jnp.float32),
        mesh=plsc.VectorSubcoreMesh(
            core_axis_name="c", subcore_axis_name="s",
            num_cores=_NC, num_subcores=_NS),
        scratch_types=[
            pltpu.VMEM((_CHUNK,), jnp.int32),
            pltpu.VMEM((_CHUNK,), jnp.int32),
            pltpu.VMEM((_CHUNK, _H), jnp.float32),
            pltpu.VMEM((_CHUNK, _H), jnp.float32),
            pltpu.VMEM((_BPW, _H), jnp.float32),
            pltpu.VMEM((_BPW,), jnp.int32),
            pltpu.SemaphoreType.DMA,
            pltpu.SemaphoreType.DMA,
        ],
    )


def kernel(old_states, new_state, prev_state, old_norms, w1):
    l2, idx, mix, dst = pl.pallas_call(
        _norm_body,
        out_shape=[
            jax.ShapeDtypeStruct((_B, 1), jnp.float32),
            jax.ShapeDtypeStruct((_B, _S), jnp.int32),
            jax.ShapeDtypeStruct((_B, _H), jnp.float32),
            jax.ShapeDtypeStruct((_NW, _BPW), jnp.int32),
        ],
    )(prev_state, w1, new_state, old_norms, old_states[:, 0, :])

    out_flat = _sc_reorder()(
        old_states.reshape(_B * _S, _H), idx.reshape(_B * _S), mix, dst)
    return out_flat.reshape(_B, _S, _H), l2


# speedup vs baseline: 3.9662x; 1.0083x over previous
import functools

import jax
import jax.numpy as jnp
from jax import lax
from jax.experimental import pallas as pl
from jax.experimental.pallas import tpu as pltpu
from jax.experimental.pallas import tpu_sc as plsc

_B = 64
_S = 64
_H = 2048
_NC = 2
_NS = 16
_NW = _NC * _NS
_BPW = _B // _NW
_RPW = (_B * _S) // _NW
_CHUNK = 16
_NT = _RPW // _CHUNK


def _norm_body(prev_ref, w1_ref, new_ref, norms_ref, old0_ref,
               l2_ref, idx_ref, mix_ref, dst_ref):
    pred = jnp.tanh(
        lax.dot_general(
            prev_ref[...], w1_ref[...], (((1,), (0,)), ((), ())),
            preferred_element_type=jnp.float32,
            precision=lax.Precision.DEFAULT,
        )
    )
    diff = pred - new_ref[...]
    l2 = jnp.sqrt(jnp.sum(diff * diff, axis=1, keepdims=True))
    n0 = norms_ref[:, 0:1]
    rep = l2 > n0
    v = jnp.where(rep, l2, n0)
    k = jnp.sum((norms_ref[:, 1:] < v).astype(jnp.int32), axis=1, keepdims=True)
    l2_ref[...] = l2
    j = lax.broadcasted_iota(jnp.int32, (_B, _S), 1)
    brow = lax.broadcasted_iota(jnp.int32, (_B, _S), 0) * _S
    src = brow + j + (j < k).astype(jnp.int32)
    idx_ref[...] = jnp.where(j == k, brow, src)
    mix_ref[...] = jnp.where(rep, new_ref[...], old0_ref[...])
    dst_ref[...] = (brow[:, 0:1] + k).reshape(_NW, _BPW)


_NBUF = 3


def _sc_reorder_body(states_ref, idx_ref, mix_ref, dst_ref, out_ref,
                     idx0, idx1, idx2, buf0, buf1, buf2, mixbuf, didx,
                     g0, g1, g2, s0, s1, s2):
    wid = lax.axis_index("s") * _NC + lax.axis_index("c")
    idxs = (idx0, idx1, idx2)
    bufs = (buf0, buf1, buf2)
    gsems = (g0, g1, g2)
    ssems = (s0, s1, s2)
    gh = [None] * _NBUF
    sh = [None] * _NBUF

    def start_gather(t):
        base = wid * _RPW + t * _CHUNK
        pltpu.sync_copy(idx_ref.at[pl.ds(base, _CHUNK)], idxs[t % _NBUF])
        gh[t % _NBUF] = pltpu.async_copy(
            states_ref.at[idxs[t % _NBUF]], bufs[t % _NBUF], gsems[t % _NBUF])

    def start_scatter(t):
        base = wid * _RPW + t * _CHUNK
        gh[t % _NBUF].wait()
        sh[t % _NBUF] = pltpu.async_copy(
            bufs[t % _NBUF], out_ref.at[pl.ds(base, _CHUNK)], ssems[t % _NBUF])

    for t in range(_NT):
        if t >= _NBUF:
            sh[t % _NBUF].wait()
        start_gather(t)
        if t >= 1:
            start_scatter(t - 1)
    start_scatter(_NT - 1)
    for t in range(_NT - _NBUF + 1, _NT + 1):
        sh[t % _NBUF].wait()

    pltpu.sync_copy(mix_ref.at[pl.ds(wid * _BPW, _BPW)], mixbuf)
    pltpu.sync_copy(dst_ref.at[wid], didx)
    pltpu.sync_copy(mixbuf, out_ref.at[didx])


@functools.cache
def _sc_reorder():
    return pl.kernel(
        _sc_reorder_body,
        out_type=jax.ShapeDtypeStruct((_B * _S, _H), jnp.float32),
        mesh=plsc.VectorSubcoreMesh(
            core_axis_name="c", subcore_axis_name="s",
            num_cores=_NC, num_subcores=_NS),
        scratch_types=[
            pltpu.VMEM((_CHUNK,), jnp.int32),
            pltpu.VMEM((_CHUNK,), jnp.int32),
            pltpu.VMEM((_CHUNK,), jnp.int32),
            pltpu.VMEM((_CHUNK, _H), jnp.float32),
            pltpu.VMEM((_CHUNK, _H), jnp.float32),
            pltpu.VMEM((_CHUNK, _H), jnp.float32),
            pltpu.VMEM((_BPW, _H), jnp.float32),
            pltpu.VMEM((_BPW,), jnp.int32),
            pltpu.SemaphoreType.DMA,
            pltpu.SemaphoreType.DMA,
            pltpu.SemaphoreType.DMA,
            pltpu.SemaphoreType.DMA,
            pltpu.SemaphoreType.DMA,
            pltpu.SemaphoreType.DMA,
        ],
    )


def kernel(old_states, new_state, prev_state, old_norms, w1):
    l2, idx, mix, dst = pl.pallas_call(
        _norm_body,
        out_shape=[
            jax.ShapeDtypeStruct((_B, 1), jnp.float32),
            jax.ShapeDtypeStruct((_B, _S), jnp.int32),
            jax.ShapeDtypeStruct((_B, _H), jnp.float32),
            jax.ShapeDtypeStruct((_NW, _BPW), jnp.int32),
        ],
    )(prev_state, w1, new_state, old_norms, old_states[:, 0, :])

    out_flat = _sc_reorder()(
        old_states.reshape(_B * _S, _H), idx.reshape(_B * _S), mix, dst)
    return out_flat.reshape(_B, _S, _H), l2
